# Initial kernel scaffold; baseline (speedup 1.0000x reference)
import functools

import jax
import jax.numpy as jnp
from jax import lax
from jax.experimental import pallas as pl
from jax.experimental.pallas import tpu as pltpu
from jax.experimental.pallas import tpu_sc as plsc

NSK = 100
E = 64
NT = 225
FL = 3
MC = 4
B, L = 8, 100
NTOK = B * L
PER = 176
ZERO = 175
NW = 32
TPW = NTOK // NW
SLOTS = 32
NCHUNK = 7
SK_BLK = 10


def _t_body(wt_ref, emb_ref, wct_ref, out_ref):
    wct = wct_ref[...]
    for ss in range(SK_BLK):
        a = wt_ref[ss]
        es = wct * emb_ref[pl.ds(ss, 1), :]
        temb = jnp.dot(es, a[:E, :], preferred_element_type=jnp.float32)
        tfull = temb + a[E:, :]
        top = tfull[0:25] + tfull[25:50] + tfull[50:75]
        out_ref[ss] = jnp.concatenate(
            [top, tfull[75:225], jnp.zeros((1, E), jnp.float32)], axis=0)


_t_precompute = pl.pallas_call(
    _t_body,
    grid=(NSK // SK_BLK,),
    in_specs=[
        pl.BlockSpec((SK_BLK, E + NT, E), lambda i: (i, 0, 0)),
        pl.BlockSpec((SK_BLK, E), lambda i: (i, 0)),
        pl.BlockSpec((NT, E), lambda i: (0, 0)),
    ],
    out_specs=pl.BlockSpec((SK_BLK, PER, E), lambda i: (i, 0, 0)),
    out_shape=jax.ShapeDtypeStruct((NSK, PER, E), jnp.float32),
)


def _idx_body(c_ref, r_ref, p_ref, a_ref, out_ref):
    c = c_ref[...]
    r = r_ref[...]
    p = p_ref[...]
    a = a_ref[...]
    col = 0
    for f in range(MC):
        cf = c[:, f:f + 1]
        win = None
        for g in range(f + 1, MC):
            neq = cf != c[:, g:g + 1]
            win = neq if win is None else (win & neq)
        base = cf * PER
        seven = [base + r[:, f:f + 1]]
        for j in range(FL):
            seven.append(base + 25 + 25 * j + p[:, FL * f + j:FL * f + j + 1])
        for j in range(FL):
            seven.append(base + 100 + 25 * j + a[:, FL * f + j:FL * f + j + 1])
        for q in seven:
            out_ref[:, col:col + 1] = q if win is None else jnp.where(win, q, ZERO)
            col += 1
    pad = jnp.full((NTOK, 1), ZERO, jnp.int32)
    for _ in range(SLOTS - 7 * MC):
        out_ref[:, col:col + 1] = pad
        col += 1


_build_idx = pl.pallas_call(
    _idx_body,
    out_shape=jax.ShapeDtypeStruct((NTOK, SLOTS), jnp.int32),
)


def _sc_body(t_hbm, idx_hbm, bias_hbm, out_hbm, idx_v, rows_v, bias_v, out_v, sem):
    wid = lax.axis_index("s") * 2 + lax.axis_index("c")
    pltpu.sync_copy(idx_hbm.at[wid], idx_v)
    pltpu.sync_copy(bias_hbm, bias_v)
    copies = [
        pltpu.async_copy(t_hbm.at[idx_v.at[j]], rows_v.at[pl.ds(j * 128, 128)], sem)
        for j in range(NCHUNK)
    ]
    for cp in copies:
        cp.wait()

    def body(t, carry):
        accs = [bias_v[pl.ds(k * 16, 16)] for k in range(E // 16)]
        base = t * SLOTS
        for j in range(7 * MC):
            row = base + j
            for k in range(E // 16):
                accs[k] = accs[k] + rows_v[row, pl.ds(k * 16, 16)]
        for k in range(E // 16):
            out_v[t, pl.ds(k * 16, 16)] = accs[k]
        return carry

    lax.fori_loop(0, TPW, body, 0)
    pltpu.sync_copy(out_v, out_hbm.at[wid])


_sc_gather = functools.partial(
    pl.kernel,
    out_type=jax.ShapeDtypeStruct((NW, TPW, E), jnp.float32),
    mesh=plsc.VectorSubcoreMesh(
        core_axis_name="c", subcore_axis_name="s", num_cores=2, num_subcores=16),
    scratch_types=[
        pltpu.VMEM((NCHUNK, 128), jnp.int32),
        pltpu.VMEM((NCHUNK * 128, E), jnp.float32),
        pltpu.VMEM((E,), jnp.float32),
        pltpu.VMEM((TPW, E), jnp.float32),
        pltpu.SemaphoreType.DMA,
    ],
)(_sc_body)


@jax.jit
def kernel(concepts, rgaps, pcounts, acounts, emb_table_skill, W_cemb, W_pre, b_pre):
    c = concepts.reshape(NTOK, MC).astype(jnp.int32)
    r = rgaps.reshape(NTOK, MC).astype(jnp.int32)
    p = pcounts.reshape(NTOK, MC * FL).astype(jnp.int32)
    a = acounts.reshape(NTOK, MC * FL).astype(jnp.int32)

    wt = jnp.transpose(W_pre.reshape(E, NSK, E + NT), (1, 2, 0))
    table = _t_precompute(wt, emb_table_skill, W_cemb.T)

    idx = _build_idx(c, r, p, a)
    idx = idx.reshape(NW, TPW * SLOTS)
    idx = jnp.concatenate(
        [idx, jnp.full((NW, NCHUNK * 128 - TPW * SLOTS), ZERO, jnp.int32)], axis=1
    ).reshape(NW, NCHUNK, 128)

    out = _sc_gather(table.reshape(NSK * PER, E), idx, b_pre)
    return out.reshape(B, L, E)


# speedup vs baseline: 5.9186x; 5.9186x over previous
import functools

import jax
import jax.numpy as jnp
from jax import lax
from jax.experimental import pallas as pl
from jax.experimental.pallas import tpu as pltpu
from jax.experimental.pallas import tpu_sc as plsc

NSK = 100
E = 64
NT = 225
FL = 3
MC = 4
B, L = 8, 100
NTOK = B * L
PER = 176
ZERO = 175
NW = 32
TPW = NTOK // NW
SLOTS = 32
NCHUNK = 7
SK_BLK = 10


def _t_body(wt_ref, emb_ref, wct_ref, out_ref):
    wct = wct_ref[...]
    for ss in range(SK_BLK):
        a = wt_ref[ss]
        es = wct * emb_ref[0, pl.ds(ss, 1), :]
        temb = jnp.dot(es, a[:E, :], preferred_element_type=jnp.float32)
        tfull = temb + a[E:, :]
        top = tfull[0:25] + tfull[25:50] + tfull[50:75]
        out_ref[ss] = jnp.concatenate(
            [top, tfull[75:225], jnp.zeros((1, E), jnp.float32)], axis=0)


_t_precompute = pl.pallas_call(
    _t_body,
    grid=(NSK // SK_BLK,),
    in_specs=[
        pl.BlockSpec((SK_BLK, E + NT, E), lambda i: (i, 0, 0)),
        pl.BlockSpec((1, SK_BLK, E), lambda i: (i, 0, 0)),
        pl.BlockSpec((NT, E), lambda i: (0, 0)),
    ],
    out_specs=pl.BlockSpec((SK_BLK, PER, E), lambda i: (i, 0, 0)),
    out_shape=jax.ShapeDtypeStruct((NSK, PER, E), jnp.float32),
)


def _idx_body(c_ref, r_ref, p_ref, a_ref, out_ref):
    c = c_ref[...]
    r = r_ref[...]
    p = p_ref[...]
    a = a_ref[...]
    col = 0
    for f in range(MC):
        cf = c[:, f:f + 1]
        win = None
        for g in range(f + 1, MC):
            neq = cf != c[:, g:g + 1]
            win = neq if win is None else (win & neq)
        base = cf * PER
        seven = [base + r[:, f:f + 1]]
        for j in range(FL):
            seven.append(base + 25 + 25 * j + p[:, FL * f + j:FL * f + j + 1])
        for j in range(FL):
            seven.append(base + 100 + 25 * j + a[:, FL * f + j:FL * f + j + 1])
        for q in seven:
            out_ref[:, col:col + 1] = q if win is None else jnp.where(win, q, ZERO)
            col += 1
    pad = jnp.full((NTOK, 1), ZERO, jnp.int32)
    for _ in range(SLOTS - 7 * MC):
        out_ref[:, col:col + 1] = pad
        col += 1


_build_idx = pl.pallas_call(
    _idx_body,
    out_shape=jax.ShapeDtypeStruct((NTOK, SLOTS), jnp.int32),
)


def _sc_body(t_hbm, idx_hbm, bias_hbm, out_hbm, idx_v, rows_v, bias_v, out_v, sem):
    wid = lax.axis_index("s") * 2 + lax.axis_index("c")
    pltpu.sync_copy(idx_hbm.at[wid], idx_v)
    pltpu.sync_copy(bias_hbm, bias_v)
    copies = [
        pltpu.async_copy(t_hbm.at[idx_v.at[j]], rows_v.at[pl.ds(j * 128, 128)], sem)
        for j in range(NCHUNK)
    ]
    for cp in copies:
        cp.wait()

    def body(t, carry):
        accs = [bias_v[pl.ds(k * 16, 16)] for k in range(E // 16)]
        base = t * SLOTS
        for j in range(7 * MC):
            row = base + j
            for k in range(E // 16):
                accs[k] = accs[k] + rows_v[row, pl.ds(k * 16, 16)]
        for k in range(E // 16):
            out_v[t, pl.ds(k * 16, 16)] = accs[k]
        return carry

    lax.fori_loop(0, TPW, body, 0)
    pltpu.sync_copy(out_v, out_hbm.at[wid])


@functools.cache
def _sc_gather_fn():
    return functools.partial(
        pl.kernel,
        out_type=jax.ShapeDtypeStruct((NW, TPW, E), jnp.float32),
        mesh=plsc.VectorSubcoreMesh(
            core_axis_name="c", subcore_axis_name="s", num_cores=2, num_subcores=16),
        scratch_types=[
            pltpu.VMEM((NCHUNK, 128), jnp.int32),
            pltpu.VMEM((NCHUNK * 128, E), jnp.float32),
            pltpu.VMEM((E,), jnp.float32),
            pltpu.VMEM((TPW, E), jnp.float32),
            pltpu.SemaphoreType.DMA,
        ],
        compiler_params=pltpu.CompilerParams(use_tc_tiling_on_sc=False),
    )(_sc_body)


@jax.jit
def kernel(concepts, rgaps, pcounts, acounts, emb_table_skill, W_cemb, W_pre, b_pre):
    c = concepts.reshape(NTOK, MC).astype(jnp.int32)
    r = rgaps.reshape(NTOK, MC).astype(jnp.int32)
    p = pcounts.reshape(NTOK, MC * FL).astype(jnp.int32)
    a = acounts.reshape(NTOK, MC * FL).astype(jnp.int32)

    wt = jnp.transpose(W_pre.reshape(E, NSK, E + NT), (1, 2, 0))
    emb3 = emb_table_skill.reshape(NSK // SK_BLK, SK_BLK, E)
    table = _t_precompute(wt, emb3, W_cemb.T)

    idx = _build_idx(c, r, p, a)
    idx = idx.reshape(NW, TPW * SLOTS)
    idx = jnp.concatenate(
        [idx, jnp.full((NW, NCHUNK * 128 - TPW * SLOTS), ZERO, jnp.int32)], axis=1
    ).reshape(NW, NCHUNK, 128)

    out = _sc_gather_fn()(table.reshape(NSK * PER, E), idx, b_pre)
    return out.reshape(B, L, E)


# speedup vs baseline: 5.9448x; 1.0044x over previous
import functools

import jax
import jax.numpy as jnp
from jax import lax
from jax.experimental import pallas as pl
from jax.experimental.pallas import tpu as pltpu
from jax.experimental.pallas import tpu_sc as plsc

NSK = 100
E = 64
NT = 225
FL = 3
MC = 4
B, L = 8, 100
NTOK = B * L
PER = 176
ZERO = 175
NW = 32
TPW = NTOK // NW
TPAD = 32
SLOTS = 7 * MC
SK_BLK = 10


def _t_body(wt_ref, emb_ref, wct_ref, out_ref):
    wct = wct_ref[...]
    for ss in range(SK_BLK):
        a = wt_ref[ss]
        es = wct * emb_ref[0, pl.ds(ss, 1), :]
        temb = jnp.dot(es, a[:E, :], preferred_element_type=jnp.float32)
        tfull = temb + a[E:, :]
        top = tfull[0:25] + tfull[25:50] + tfull[50:75]
        out_ref[ss] = jnp.concatenate(
            [top, tfull[75:225], jnp.zeros((1, E), jnp.float32)], axis=0)


_t_precompute = pl.pallas_call(
    _t_body,
    grid=(NSK // SK_BLK,),
    in_specs=[
        pl.BlockSpec((SK_BLK, E + NT, E), lambda i: (i, 0, 0)),
        pl.BlockSpec((1, SK_BLK, E), lambda i: (i, 0, 0)),
        pl.BlockSpec((NT, E), lambda i: (0, 0)),
    ],
    out_specs=pl.BlockSpec((SK_BLK, PER, E), lambda i: (i, 0, 0)),
    out_shape=jax.ShapeDtypeStruct((NSK, PER, E), jnp.float32),
)


def _idx_body(c_ref, r_ref, p_ref, a_ref, out_ref):
    c = c_ref[...]
    r = r_ref[...]
    p = p_ref[...]
    a = a_ref[...]
    col = 0
    for f in range(MC):
        cf = c[:, f:f + 1]
        win = None
        for g in range(f + 1, MC):
            neq = cf != c[:, g:g + 1]
            win = neq if win is None else (win & neq)
        base = cf * PER
        seven = [base + r[:, f:f + 1]]
        for j in range(FL):
            seven.append(base + 25 + 25 * j + p[:, FL * f + j:FL * f + j + 1])
        for j in range(FL):
            seven.append(base + 100 + 25 * j + a[:, FL * f + j:FL * f + j + 1])
        for q in seven:
            out_ref[:, col:col + 1] = q if win is None else jnp.where(win, q, ZERO)
            col += 1


_build_idx = pl.pallas_call(
    _idx_body,
    out_shape=jax.ShapeDtypeStruct((NTOK, SLOTS), jnp.int32),
)


def _sc_body(t_hbm, idx_hbm, bias_hbm, out_hbm, idx_v, bias_v, out_v, sem):
    wid = lax.axis_index("s") * 2 + lax.axis_index("c")
    pltpu.sync_copy(idx_hbm.at[wid], idx_v)
    pltpu.sync_copy(bias_hbm, bias_v)

    bias_regs = [bias_v[pl.ds(k * 16, 16)] for k in range(E // 16)]

    def initb(t, carry):
        for k in range(E // 16):
            out_v[t, pl.ds(k * 16, 16)] = bias_regs[k]
        return carry

    lax.fori_loop(0, TPAD, initb, 0)

    copies = [
        pltpu.async_copy(t_hbm.at[idx_v.at[j]], out_v, sem, add=True)
        for j in range(SLOTS)
    ]
    for cp in copies:
        cp.wait()
    pltpu.sync_copy(out_v, out_hbm.at[wid])


@functools.cache
def _sc_gather_fn():
    return functools.partial(
        pl.kernel,
        out_type=jax.ShapeDtypeStruct((NW, TPAD, E), jnp.float32),
        mesh=plsc.VectorSubcoreMesh(
            core_axis_name="c", subcore_axis_name="s", num_cores=2, num_subcores=16),
        scratch_types=[
            pltpu.VMEM((SLOTS, TPAD), jnp.int32),
            pltpu.VMEM((E,), jnp.float32),
            pltpu.VMEM((TPAD, E), jnp.float32),
            pltpu.SemaphoreType.DMA,
        ],
        compiler_params=pltpu.CompilerParams(use_tc_tiling_on_sc=False),
    )(_sc_body)


@jax.jit
def kernel(concepts, rgaps, pcounts, acounts, emb_table_skill, W_cemb, W_pre, b_pre):
    c = concepts.reshape(NTOK, MC).astype(jnp.int32)
    r = rgaps.reshape(NTOK, MC).astype(jnp.int32)
    p = pcounts.reshape(NTOK, MC * FL).astype(jnp.int32)
    a = acounts.reshape(NTOK, MC * FL).astype(jnp.int32)

    wt = jnp.transpose(W_pre.reshape(E, NSK, E + NT), (1, 2, 0))
    emb3 = emb_table_skill.reshape(NSK // SK_BLK, SK_BLK, E)
    table = _t_precompute(wt, emb3, W_cemb.T)

    idx = _build_idx(c, r, p, a)
    idx = idx.reshape(NW, TPW, SLOTS).transpose(0, 2, 1)
    idx = jnp.concatenate(
        [idx, jnp.full((NW, SLOTS, TPAD - TPW), ZERO, jnp.int32)], axis=2)

    out = _sc_gather_fn()(table.reshape(NSK * PER, E), idx, b_pre)
    return out[:, :TPW, :].reshape(B, L, E)


# speedup vs baseline: 14.7269x; 2.4773x over previous
import functools

import jax
import jax.numpy as jnp
from jax import lax
from jax.experimental import pallas as pl
from jax.experimental.pallas import tpu as pltpu
from jax.experimental.pallas import tpu_sc as plsc

NSK = 100
E = 64
NT = 225
FL = 3
MC = 4
B, L = 8, 100
NTOK = B * L
PER = 176
ZERO = 175
NW = 32
TPW = NTOK // NW
TPAD = 32
SLOTS = 7 * MC
SK_BLK = 10


def _t_body(wt_ref, emb_ref, wct_ref, out_ref):
    wct = wct_ref[...]
    for ss in range(SK_BLK):
        a = wt_ref[ss]
        es = wct * emb_ref[0, pl.ds(ss, 1), :]
        temb = jnp.dot(es, a[:E, :], preferred_element_type=jnp.float32)
        tfull = temb + a[E:, :]
        top = tfull[0:25] + tfull[25:50] + tfull[50:75]
        out_ref[ss] = jnp.concatenate(
            [top, tfull[75:225], jnp.zeros((1, E), jnp.float32)], axis=0)


_t_precompute = pl.pallas_call(
    _t_body,
    grid=(NSK // SK_BLK,),
    in_specs=[
        pl.BlockSpec((SK_BLK, E + NT, E), lambda i: (i, 0, 0)),
        pl.BlockSpec((1, SK_BLK, E), lambda i: (i, 0, 0)),
        pl.BlockSpec((NT, E), lambda i: (0, 0)),
    ],
    out_specs=pl.BlockSpec((SK_BLK, PER, E), lambda i: (i, 0, 0)),
    out_shape=jax.ShapeDtypeStruct((NSK, PER, E), jnp.float32),
)


def _idx_body(c_ref, r_ref, p_ref, a_ref, out_ref):
    c = c_ref[...]
    r = r_ref[...]
    p = p_ref[...]
    a = a_ref[...]
    col = 0
    for f in range(MC):
        cf = c[:, f:f + 1]
        win = None
        for g in range(f + 1, MC):
            neq = cf != c[:, g:g + 1]
            win = neq if win is None else (win & neq)
        base = cf * PER
        seven = [base + r[:, f:f + 1]]
        for j in range(FL):
            seven.append(base + 25 + 25 * j + p[:, FL * f + j:FL * f + j + 1])
        for j in range(FL):
            seven.append(base + 100 + 25 * j + a[:, FL * f + j:FL * f + j + 1])
        for q in seven:
            out_ref[:, col:col + 1] = q if win is None else jnp.where(win, q, ZERO)
            col += 1


_build_idx = pl.pallas_call(
    _idx_body,
    out_shape=jax.ShapeDtypeStruct((NTOK, SLOTS), jnp.int32),
)


_STAGE = (NSK * PER) // 16


def _sc_body(t_hbm, idx_hbm, bias_hbm, out_hbm, idx_v, bias_v, out_v, shared, sem):
    sid = lax.axis_index("s")
    wid = sid * 2 + lax.axis_index("c")
    pltpu.sync_copy(idx_hbm.at[wid], idx_v)
    pltpu.sync_copy(bias_hbm, bias_v)

    pltpu.sync_copy(t_hbm.at[pl.ds(sid * _STAGE, _STAGE)],
                    shared.at[pl.ds(sid * _STAGE, _STAGE)])

    bias_regs = [bias_v[pl.ds(k * 16, 16)] for k in range(E // 16)]

    def initb(t, carry):
        for k in range(E // 16):
            out_v[t, pl.ds(k * 16, 16)] = bias_regs[k]
        return carry

    lax.fori_loop(0, TPAD, initb, 0)
    plsc.subcore_barrier()

    copies = [
        pltpu.async_copy(shared.at[idx_v.at[j]], out_v, sem, add=True)
        for j in range(SLOTS)
    ]
    for cp in copies:
        cp.wait()
    pltpu.sync_copy(out_v, out_hbm.at[wid])


@functools.cache
def _sc_gather_fn():
    return functools.partial(
        pl.kernel,
        out_type=jax.ShapeDtypeStruct((NW, TPAD, E), jnp.float32),
        mesh=plsc.VectorSubcoreMesh(
            core_axis_name="c", subcore_axis_name="s", num_cores=2, num_subcores=16),
        scratch_types=[
            pltpu.VMEM((SLOTS, TPAD), jnp.int32),
            pltpu.VMEM((E,), jnp.float32),
            pltpu.VMEM((TPAD, E), jnp.float32),
            pltpu.VMEM_SHARED((NSK * PER, E), jnp.float32),
            pltpu.SemaphoreType.DMA,
        ],
        compiler_params=pltpu.CompilerParams(use_tc_tiling_on_sc=False),
    )(_sc_body)


@jax.jit
def kernel(concepts, rgaps, pcounts, acounts, emb_table_skill, W_cemb, W_pre, b_pre):
    c = concepts.reshape(NTOK, MC).astype(jnp.int32)
    r = rgaps.reshape(NTOK, MC).astype(jnp.int32)
    p = pcounts.reshape(NTOK, MC * FL).astype(jnp.int32)
    a = acounts.reshape(NTOK, MC * FL).astype(jnp.int32)

    wt = jnp.transpose(W_pre.reshape(E, NSK, E + NT), (1, 2, 0))
    emb3 = emb_table_skill.reshape(NSK // SK_BLK, SK_BLK, E)
    table = _t_precompute(wt, emb3, W_cemb.T)

    idx = _build_idx(c, r, p, a)
    idx = idx.reshape(NW, TPW, SLOTS).transpose(0, 2, 1)
    idx = jnp.concatenate(
        [idx, jnp.full((NW, SLOTS, TPAD - TPW), ZERO, jnp.int32)], axis=2)

    out = _sc_gather_fn()(table.reshape(NSK * PER, E), idx, b_pre)
    return out[:, :TPW, :].reshape(B, L, E)


# speedup vs baseline: 16.3550x; 1.1106x over previous
import functools

import jax
import jax.numpy as jnp
from jax import lax
from jax.experimental import pallas as pl
from jax.experimental.pallas import tpu as pltpu
from jax.experimental.pallas import tpu_sc as plsc

NSK = 100
E = 64
NT = 225
FL = 3
MC = 4
B, L = 8, 100
NTOK = B * L
PER = 176
ZERO = 175
NW = 32
TPW = NTOK // NW
TPAD = 32
SLOTS = 7 * MC
SK_BLK = 10
_STAGE = (NSK * PER) // 16


def _t_body(wp_ref, emb_ref, wc_ref, out_ref):
    wc = wc_ref[...]
    ii = jnp.arange(NT, dtype=jnp.int32)
    eye = (ii[:, None] == ii[None, :]).astype(jnp.float32)
    wc_aug = jnp.concatenate([wc, eye], axis=0)
    ones_row = jnp.ones((1, NT), jnp.float32)
    for ss in range(SK_BLK):
        aa = wp_ref[:, 0, ss, :]
        scale = jnp.concatenate(
            [emb_ref[0, pl.ds(ss, 1), :], ones_row], axis=1)
        b_aug = aa * scale
        tfull = lax.dot_general(
            wc_aug, b_aug, (((0,), (1,)), ((), ())),
            preferred_element_type=jnp.float32)
        top = tfull[0:25] + tfull[25:50] + tfull[50:75]
        out_ref[ss] = jnp.concatenate(
            [top, tfull[75:225], jnp.zeros((1, E), jnp.float32)], axis=0)


_t_precompute = pl.pallas_call(
    _t_body,
    grid=(NSK // SK_BLK,),
    in_specs=[
        pl.BlockSpec((E, 1, SK_BLK, E + NT), lambda i: (0, i, 0, 0)),
        pl.BlockSpec((1, SK_BLK, E), lambda i: (i, 0, 0)),
        pl.BlockSpec((E, NT), lambda i: (0, 0)),
    ],
    out_specs=pl.BlockSpec((SK_BLK, PER, E), lambda i: (i, 0, 0)),
    out_shape=jax.ShapeDtypeStruct((NSK, PER, E), jnp.float32),
)


def _sc_body(meta_hbm, t_hbm, bias_hbm, out_hbm,
             meta_v, idx_v, bias_v, out_v, shared, sem, sem2):
    sid = lax.axis_index("s")
    wid = sid * 2 + lax.axis_index("c")

    stage = pltpu.async_copy(t_hbm.at[pl.ds(sid * _STAGE, _STAGE)],
                             shared.at[pl.ds(sid * _STAGE, _STAGE)], sem2)

    pltpu.sync_copy(meta_hbm.at[wid], meta_v)
    pltpu.sync_copy(bias_hbm, bias_v)

    for chunk in range(2):
        sl = pl.ds(chunk * 16, 16)
        cs = [meta_v[f, sl] for f in range(MC)]
        rs = [meta_v[MC + f, sl] for f in range(MC)]
        ps = [meta_v[2 * MC + k, sl] for k in range(MC * FL)]
        asv = [meta_v[2 * MC + MC * FL + k, sl] for k in range(MC * FL)]
        j = 0
        for f in range(MC):
            win = None
            for g in range(f + 1, MC):
                neq = cs[f] != cs[g]
                win = neq if win is None else win & neq
            base = cs[f] * PER
            seven = [base + rs[f]]
            for q in range(FL):
                seven.append(base + 25 + 25 * q + ps[FL * f + q])
            for q in range(FL):
                seven.append(base + 100 + 25 * q + asv[FL * f + q])
            for vec in seven:
                v = vec if win is None else jnp.where(win, vec, ZERO)
                idx_v[j, pl.ds(chunk * 16, 16)] = v
                j += 1

    bias_regs = [bias_v[pl.ds(k * 16, 16)] for k in range(E // 16)]

    def initb(t, carry):
        for k in range(E // 16):
            out_v[t, pl.ds(k * 16, 16)] = bias_regs[k]
        return carry

    lax.fori_loop(0, TPAD, initb, 0)

    stage.wait()
    plsc.subcore_barrier()

    copies = [
        pltpu.async_copy(shared.at[idx_v.at[j]], out_v, sem, add=True)
        for j in range(SLOTS)
    ]
    for cp in copies:
        cp.wait()
    pltpu.sync_copy(out_v.at[pl.ds(0, TPW)], out_hbm.at[pl.ds(wid * TPW, TPW)])


@functools.cache
def _sc_gather_fn():
    return functools.partial(
        pl.kernel,
        out_type=jax.ShapeDtypeStruct((NTOK, E), jnp.float32),
        mesh=plsc.VectorSubcoreMesh(
            core_axis_name="c", subcore_axis_name="s", num_cores=2, num_subcores=16),
        scratch_types=[
            pltpu.VMEM((2 * MC + 2 * MC * FL, TPAD), jnp.int32),
            pltpu.VMEM((SLOTS, TPAD), jnp.int32),
            pltpu.VMEM((E,), jnp.float32),
            pltpu.VMEM((TPAD, E), jnp.float32),
            pltpu.VMEM_SHARED((NSK * PER, E), jnp.float32),
            pltpu.SemaphoreType.DMA,
            pltpu.SemaphoreType.DMA,
        ],
        compiler_params=pltpu.CompilerParams(use_tc_tiling_on_sc=False),
    )(_sc_body)


@jax.jit
def kernel(concepts, rgaps, pcounts, acounts, emb_table_skill, W_cemb, W_pre, b_pre):
    cw = concepts.reshape(NW, TPW, MC).astype(jnp.int32).transpose(0, 2, 1)
    rw = rgaps.reshape(NW, TPW, MC).astype(jnp.int32).transpose(0, 2, 1)
    pw = pcounts.reshape(NW, TPW, MC * FL).astype(jnp.int32).transpose(0, 2, 1)
    aw = acounts.reshape(NW, TPW, MC * FL).astype(jnp.int32).transpose(0, 2, 1)
    meta = jnp.concatenate([cw, rw, pw, aw], axis=1)
    meta = jnp.pad(meta, ((0, 0), (0, 0), (0, TPAD - TPW)))

    wp4 = W_pre.reshape(E, NSK // SK_BLK, SK_BLK, E + NT)
    emb3 = emb_table_skill.reshape(NSK // SK_BLK, SK_BLK, E)
    table = _t_precompute(wp4, emb3, W_cemb)

    out = _sc_gather_fn()(meta, table.reshape(NSK * PER, E), b_pre)
    return out.reshape(B, L, E)
